# baseline (speedup 1.0000x reference)
import functools

import jax
import jax.numpy as jnp
from jax import lax
from jax.experimental import pallas as pl
from jax.experimental.pallas import tpu as pltpu
from jax.experimental.pallas import tpu_sc as plsc

N, E, EL, D, H = 10000, 320000, 320000, 128, 128

_NC, _NS, _L = 2, 16, 16
_NW = _NC * _NS
_EW = E // _NW
_CH = 80
_NCH = _EW // _CH
_ECT = E // _NS
_NCHS = _ECT // _CH
_NHALF = N // 2
_ACCR = _NHALF + 8
_TRASH = _NHALF

_MESH = plsc.VectorSubcoreMesh(core_axis_name="c", subcore_axis_name="s",
                               num_cores=_NC, num_subcores=_NS)
_SC_PARAMS = pltpu.CompilerParams(needs_layout_passes=False)


def _wid():
    return lax.axis_index("c") * _NS + lax.axis_index("s")


def _pre_body(x_ref, wpre_ref, bpre_ref, iw_ref, wih_ref, whh_ref,
              bih_ref, bhh_ref, hw_ref):
    iw = iw_ref[...]
    gi = lax.dot_general(iw, wih_ref[...], (((1,), (1,)), ((), ())),
                         preferred_element_type=jnp.float32) + bih_ref[...]
    gh = lax.dot_general(iw, whh_ref[...], (((1,), (1,)), ((), ())),
                         preferred_element_type=jnp.float32) + bhh_ref[...]
    i_r, i_z, i_n = gi[:, :H], gi[:, H:2 * H], gi[:, 2 * H:]
    h_r, h_z, h_n = gh[:, :H], gh[:, H:2 * H], gh[:, 2 * H:]
    r = jax.nn.sigmoid(i_r + h_r)
    z = jax.nn.sigmoid(i_z + h_z)
    n = jnp.tanh(i_n + r * h_n)
    W = (1.0 - z) * n + z * iw
    M = lax.dot_general(wpre_ref[...], W, (((0,), (0,)), ((), ())),
                        preferred_element_type=jnp.float32)
    v = jnp.dot(bpre_ref[...], W, preferred_element_type=jnp.float32)
    hw_ref[...] = jnp.dot(x_ref[...], M,
                          preferred_element_type=jnp.float32) + v


def _pre(x, W_pre, b_pre, iw, W_ih, W_hh, b_ih, b_hh):
    return pl.pallas_call(
        _pre_body,
        out_shape=jax.ShapeDtypeStruct((N, H), jnp.float32),
    )(x, W_pre, b_pre.reshape(1, H), iw, W_ih, W_hh,
      b_ih.reshape(1, 3 * H), b_hh.reshape(1, 3 * H))


def _scale_body(hw_ref, degp_ref, g_ref, dinv_ref, selfs_ref):
    deg = 1.0 + jnp.sum(degp_ref[...], axis=0)
    dinv = lax.rsqrt(deg)
    dinv_ref[...] = dinv[:, None]
    selfs_ref[...] = (1.0 / deg)[:, None]
    g_ref[...] = hw_ref[...] * dinv[:, None]


def _scale(hW, deg_partials):
    return pl.pallas_call(
        _scale_body,
        out_shape=(
            jax.ShapeDtypeStruct((N, H), jnp.float32),
            jax.ShapeDtypeStruct((N, 1), jnp.float32),
            jax.ShapeDtypeStruct((N, 1), jnp.float32),
        ),
    )(hW, deg_partials)


def _fin_body(acc_ref, hw_ref, dinv_ref, selfs_ref, bias_ref, w_ref,
              ha_ref, hb_ref):
    acc = jnp.concatenate([acc_ref[0, :_NHALF], acc_ref[1, :_NHALF]], axis=0)
    h = acc * dinv_ref[...] + hw_ref[...] * selfs_ref[...] + bias_ref[...]
    h = jnp.maximum(h, 0.0)
    hb_ref[...] = h
    ha_ref[...] = h * w_ref[...]


def _finalize(accs, hW, dinv, selfs, gcn_bias, w):
    return pl.pallas_call(
        _fin_body,
        out_shape=(
            jax.ShapeDtypeStruct((N, H), jnp.float32),
            jax.ShapeDtypeStruct((N, H), jnp.float32),
        ),
    )(accs, hW, dinv, selfs, gcn_bias.reshape(1, H), w.reshape(1, H))


def _deg_body(dst_hbm, out_hbm, idx_v, hist_v):
    w = _wid()
    pltpu.sync_copy(dst_hbm.at[w], idx_v)
    zeros = jnp.zeros((_L,), jnp.float32)

    def zbody(i, carry):
        hist_v[pl.ds(i * _L, _L)] = zeros
        return carry

    lax.fori_loop(0, N // _L, zbody, 0)
    ones = jnp.ones((_L,), jnp.float32)

    def cbody(i, carry):
        idx = idx_v[pl.ds(i * _L, _L)]
        plsc.addupdate_scatter(hist_v, [idx], ones)
        return carry

    lax.fori_loop(0, _EW // _L, cbody, 0)
    pltpu.sync_copy(hist_v, out_hbm.at[w])


def _deg_partials(dst2):
    return pl.kernel(
        _deg_body,
        out_type=jax.ShapeDtypeStruct((_NW, N), jnp.float32),
        mesh=_MESH,
        compiler_params=_SC_PARAMS,
        scratch_types=[
            pltpu.VMEM((_EW,), jnp.int32),
            pltpu.VMEM((N,), jnp.float32),
        ],
    )(dst2)


def _scat_body(g_hbm, src_hbm, dst_hbm, zeros_hbm, acc_out,
               sidx_v, didx_v, buf0, buf1, acc_sp, sem0, sem1):
    cid = lax.axis_index("c")
    s = lax.axis_index("s")

    start = s * 312
    pltpu.sync_copy(zeros_hbm.at[pl.ds(0, 312)], acc_sp.at[pl.ds(start, 312)])

    @pl.when(s == _NS - 1)
    def _():
        pltpu.sync_copy(zeros_hbm.at[pl.ds(0, 16)],
                        acc_sp.at[pl.ds(4992, 16)])

    plsc.subcore_barrier()

    base = cid * _NHALF
    bufs = (buf0, buf1)
    sems = (sem0, sem1)

    for hlf in range(2):
        pltpu.sync_copy(src_hbm.at[s, hlf], sidx_v)
        pltpu.sync_copy(dst_hbm.at[s, hlf], didx_v)

        def tbody(j, carry):
            for k in range(_CH // _L):
                v = didx_v[j, pl.ds(k * _L, _L)] - base
                ok = (v >= 0) & (v < _NHALF)
                didx_v[j, pl.ds(k * _L, _L)] = jnp.where(
                    ok, v, jnp.full((_L,), _TRASH, jnp.int32))
            return carry

        lax.fori_loop(0, _NCH, tbody, 0)

        pltpu.async_copy(g_hbm.at[sidx_v.at[0]], buf0, sem0)

        def lbody(j2, carry):
            for ph in range(2):
                j = j2 * 2 + ph
                cur, csem = bufs[ph], sems[ph]

                @pl.when(j + 1 < _NCH)
                def _():
                    pltpu.async_copy(g_hbm.at[sidx_v.at[j + 1]],
                                     bufs[1 - ph], sems[1 - ph])

                pltpu.make_async_copy(g_hbm.at[sidx_v.at[j]], cur,
                                      csem).wait()
                pltpu.sync_copy(cur, acc_sp.at[didx_v.at[j]], add=True)
            return carry

        lax.fori_loop(0, _NCH // 2, lbody, 0)
        jt = _NCH - 1
        pltpu.make_async_copy(g_hbm.at[sidx_v.at[jt]], buf0, sem0).wait()
        pltpu.sync_copy(buf0, acc_sp.at[didx_v.at[jt]], add=True)

    plsc.subcore_barrier()
    pltpu.sync_copy(acc_sp.at[pl.ds(start, 312)],
                    acc_out.at[cid, pl.ds(start, 312)])

    @pl.when(s == _NS - 1)
    def _():
        pltpu.sync_copy(acc_sp.at[pl.ds(4992, 16)],
                        acc_out.at[cid, pl.ds(4992, 16)])


def _scatter_accs(g, src3, dst3):
    zeros = jnp.zeros((312, H), jnp.float32)
    return pl.kernel(
        _scat_body,
        out_type=jax.ShapeDtypeStruct((_NC, _ACCR, H), jnp.float32),
        mesh=_MESH,
        compiler_params=_SC_PARAMS,
        scratch_types=[
            pltpu.VMEM((_NCH, _CH), jnp.int32),
            pltpu.VMEM((_NCH, _CH), jnp.int32),
            pltpu.VMEM((_CH, H), jnp.float32),
            pltpu.VMEM((_CH, H), jnp.float32),
            pltpu.VMEM_SHARED((_ACCR, H), jnp.float32),
            pltpu.SemaphoreType.DMA,
            pltpu.SemaphoreType.DMA,
        ],
    )(g, src3, dst3, zeros)


_SCH = 80
_SNCH = _EW // _SCH


def _score_body(ha_hbm, hb_hbm, lsrc_hbm, ldst_hbm, cvec_hbm, out_hbm,
                sidx_v, didx_v, a0, a1, b0, b1, cv_v, res_v,
                sa0, sa1, sb0, sb1):
    w = _wid()
    pltpu.sync_copy(lsrc_hbm.at[w], sidx_v)
    pltpu.sync_copy(ldst_hbm.at[w], didx_v)
    pltpu.sync_copy(cvec_hbm, cv_v)
    cval = cv_v[...][0]
    lanes = lax.iota(jnp.int32, _L)

    abufs, asems = (a0, a1), (sa0, sa1)
    bbufs, bsems = (b0, b1), (sb0, sb1)
    pltpu.async_copy(ha_hbm.at[sidx_v.at[0]], a0, sa0)
    pltpu.async_copy(hb_hbm.at[didx_v.at[0]], b0, sb0)
    zero16 = jnp.zeros((_L,), jnp.float32)

    def _chunk(j, ca, cb):
        def gbody(gi, carry):
            rows = lanes + gi * _L
            accs = [zero16, zero16, zero16, zero16]
            for d in range(H):
                cols = jnp.full((_L,), d, jnp.int32)
                va = plsc.load_gather(ca, [rows, cols])
                vb = plsc.load_gather(cb, [rows, cols])
                accs[d % 4] = accs[d % 4] + va * vb
            acc = (accs[0] + accs[1]) + (accs[2] + accs[3])
            res_v[pl.ds(j * _SCH + gi * _L, _L)] = acc + cval
            return carry

        lax.fori_loop(0, _SCH // _L, gbody, 0)

    def lbody(j2, carry):
        for ph in range(2):
            j = j2 * 2 + ph
            ca, cb = abufs[ph], bbufs[ph]
            csa, csb = asems[ph], bsems[ph]

            @pl.when(j + 1 < _SNCH)
            def _():
                pltpu.async_copy(ha_hbm.at[sidx_v.at[j + 1]],
                                 abufs[1 - ph], asems[1 - ph])
                pltpu.async_copy(hb_hbm.at[didx_v.at[j + 1]],
                                 bbufs[1 - ph], bsems[1 - ph])

            pltpu.make_async_copy(ha_hbm.at[sidx_v.at[j]], ca, csa).wait()
            pltpu.make_async_copy(hb_hbm.at[didx_v.at[j]], cb, csb).wait()
            _chunk(j, ca, cb)
        return carry

    lax.fori_loop(0, _SNCH // 2, lbody, 0)
    jt = _SNCH - 1
    pltpu.make_async_copy(ha_hbm.at[sidx_v.at[jt]], a0, sa0).wait()
    pltpu.make_async_copy(hb_hbm.at[didx_v.at[jt]], b0, sb0).wait()
    _chunk(jt, a0, b0)
    pltpu.sync_copy(res_v, out_hbm.at[pl.ds(w * _EW, _EW)])


def _score(hA, hB, lsrc3, ldst3, cvec):
    return pl.kernel(
        _score_body,
        out_type=jax.ShapeDtypeStruct((EL,), jnp.float32),
        mesh=_MESH,
        compiler_params=_SC_PARAMS,
        scratch_types=[
            pltpu.VMEM((_SNCH, _SCH), jnp.int32),
            pltpu.VMEM((_SNCH, _SCH), jnp.int32),
            pltpu.VMEM((_SCH, H), jnp.float32),
            pltpu.VMEM((_SCH, H), jnp.float32),
            pltpu.VMEM((_SCH, H), jnp.float32),
            pltpu.VMEM((_SCH, H), jnp.float32),
            pltpu.VMEM((_L,), jnp.float32),
            pltpu.VMEM((_EW,), jnp.float32),
            pltpu.SemaphoreType.DMA,
            pltpu.SemaphoreType.DMA,
            pltpu.SemaphoreType.DMA,
            pltpu.SemaphoreType.DMA,
        ],
    )(hA, hB, lsrc3, ldst3, cvec)


def kernel(x, edge_index, edge_label_index, W_pre, b_pre, initial_weight,
           W_ih, W_hh, b_ih, b_hh, gcn_bias, W_post, b_post):
    src3 = edge_index[0].astype(jnp.int32).reshape(_NS, 2, _NCH, _CH)
    dst3 = edge_index[1].astype(jnp.int32).reshape(_NS, 2, _NCH, _CH)
    dst2 = edge_index[1].astype(jnp.int32).reshape(_NW, _EW)
    lsrc3 = edge_label_index[0].astype(jnp.int32).reshape(_NW, _SNCH, _SCH)
    ldst3 = edge_label_index[1].astype(jnp.int32).reshape(_NW, _SNCH, _SCH)
    w = W_post[0] + W_post[1]
    cvec = jnp.full((_L,), b_post[0] + b_post[1], jnp.float32)

    hW = _pre(x.astype(jnp.float32), W_pre, b_pre, initial_weight,
              W_ih, W_hh, b_ih, b_hh)
    degp = _deg_partials(dst2)
    g, dinv, selfs = _scale(hW, degp)
    accs = _scatter_accs(g, src3, dst3)
    hA, hB = _finalize(accs, hW, dinv, selfs, gcn_bias, w)
    return _score(hA, hB, lsrc3, ldst3, cvec)


# speedup vs baseline: 2.6354x; 2.6354x over previous
import functools

import jax
import jax.numpy as jnp
from jax import lax
from jax.experimental import pallas as pl
from jax.experimental.pallas import tpu as pltpu
from jax.experimental.pallas import tpu_sc as plsc

N, E, EL, D, H = 10000, 320000, 320000, 128, 128

_NC, _NS, _L = 2, 16, 16
_NW = _NC * _NS
_EW = E // _NW
_CH = 80
_NCH = _EW // _CH
_ECT = E // _NS
_NCHS = _ECT // _CH
_NHALF = N // 2
_ACCR = _NHALF + 8
_TRASH = _NHALF

_MESH = plsc.VectorSubcoreMesh(core_axis_name="c", subcore_axis_name="s",
                               num_cores=_NC, num_subcores=_NS)
_SC_PARAMS = pltpu.CompilerParams(needs_layout_passes=False)


def _wid():
    return lax.axis_index("c") * _NS + lax.axis_index("s")


def _pre_body(x_ref, wpre_ref, bpre_ref, iw_ref, wih_ref, whh_ref,
              bih_ref, bhh_ref, hw_ref):
    iw = iw_ref[...]
    gi = lax.dot_general(iw, wih_ref[...], (((1,), (1,)), ((), ())),
                         preferred_element_type=jnp.float32) + bih_ref[...]
    gh = lax.dot_general(iw, whh_ref[...], (((1,), (1,)), ((), ())),
                         preferred_element_type=jnp.float32) + bhh_ref[...]
    i_r, i_z, i_n = gi[:, :H], gi[:, H:2 * H], gi[:, 2 * H:]
    h_r, h_z, h_n = gh[:, :H], gh[:, H:2 * H], gh[:, 2 * H:]
    r = jax.nn.sigmoid(i_r + h_r)
    z = jax.nn.sigmoid(i_z + h_z)
    n = jnp.tanh(i_n + r * h_n)
    W = (1.0 - z) * n + z * iw
    M = lax.dot_general(wpre_ref[...], W, (((0,), (0,)), ((), ())),
                        preferred_element_type=jnp.float32)
    v = jnp.dot(bpre_ref[...], W, preferred_element_type=jnp.float32)
    hw_ref[...] = jnp.dot(x_ref[...], M,
                          preferred_element_type=jnp.float32) + v


def _pre(x, W_pre, b_pre, iw, W_ih, W_hh, b_ih, b_hh):
    return pl.pallas_call(
        _pre_body,
        out_shape=jax.ShapeDtypeStruct((N, H), jnp.float32),
    )(x, W_pre, b_pre.reshape(1, H), iw, W_ih, W_hh,
      b_ih.reshape(1, 3 * H), b_hh.reshape(1, 3 * H))


def _scale_body(hw_ref, degp_ref, g_ref, dinv_ref, selfs_ref):
    deg = 1.0 + jnp.sum(degp_ref[...], axis=0)
    dinv = lax.rsqrt(deg)
    dinv_ref[...] = dinv[:, None]
    selfs_ref[...] = (1.0 / deg)[:, None]
    g_ref[...] = hw_ref[...] * dinv[:, None]


def _scale(hW, deg_partials):
    return pl.pallas_call(
        _scale_body,
        out_shape=(
            jax.ShapeDtypeStruct((N, H), jnp.float32),
            jax.ShapeDtypeStruct((N, 1), jnp.float32),
            jax.ShapeDtypeStruct((N, 1), jnp.float32),
        ),
    )(hW, deg_partials)


def _fin_body(acc_ref, hw_ref, dinv_ref, selfs_ref, bias_ref, w_ref,
              ha_ref, hb_ref):
    acc = jnp.concatenate([acc_ref[0, :_NHALF], acc_ref[1, :_NHALF]], axis=0)
    h = acc * dinv_ref[...] + hw_ref[...] * selfs_ref[...] + bias_ref[...]
    h = jnp.maximum(h, 0.0)
    hb_ref[...] = h
    ha_ref[...] = h * w_ref[...]


def _finalize(accs, hW, dinv, selfs, gcn_bias, w):
    return pl.pallas_call(
        _fin_body,
        out_shape=(
            jax.ShapeDtypeStruct((N, H), jnp.float32),
            jax.ShapeDtypeStruct((N, H), jnp.float32),
        ),
    )(accs, hW, dinv, selfs, gcn_bias.reshape(1, H), w.reshape(1, H))


def _deg_body(dst_hbm, out_hbm, idx_v, hist_v):
    w = _wid()
    pltpu.sync_copy(dst_hbm.at[w], idx_v)
    zeros = jnp.zeros((_L,), jnp.float32)

    def zbody(i, carry):
        hist_v[pl.ds(i * _L, _L)] = zeros
        return carry

    lax.fori_loop(0, N // _L, zbody, 0)
    ones = jnp.ones((_L,), jnp.float32)

    def cbody(i, carry):
        idx = idx_v[pl.ds(i * _L, _L)]
        plsc.addupdate_scatter(hist_v, [idx], ones)
        return carry

    lax.fori_loop(0, _EW // _L, cbody, 0)
    pltpu.sync_copy(hist_v, out_hbm.at[w])


def _deg_partials(dst2):
    return pl.kernel(
        _deg_body,
        out_type=jax.ShapeDtypeStruct((_NW, N), jnp.float32),
        mesh=_MESH,
        compiler_params=_SC_PARAMS,
        scratch_types=[
            pltpu.VMEM((_EW,), jnp.int32),
            pltpu.VMEM((N,), jnp.float32),
        ],
    )(dst2)


def _scat_body(g_hbm, src_hbm, dst_hbm, zeros_hbm, acc_out,
               sidx_v, didx_v, buf0, buf1, acc_sp, sem0, sem1):
    cid = lax.axis_index("c")
    s = lax.axis_index("s")

    start = s * 312
    pltpu.sync_copy(zeros_hbm.at[pl.ds(0, 312)], acc_sp.at[pl.ds(start, 312)])

    @pl.when(s == _NS - 1)
    def _():
        pltpu.sync_copy(zeros_hbm.at[pl.ds(0, 16)],
                        acc_sp.at[pl.ds(4992, 16)])

    plsc.subcore_barrier()

    base = cid * _NHALF
    bufs = (buf0, buf1)
    sems = (sem0, sem1)

    for hlf in range(2):
        pltpu.sync_copy(src_hbm.at[s, hlf], sidx_v)
        pltpu.sync_copy(dst_hbm.at[s, hlf], didx_v)

        def tbody(j, carry):
            for k in range(_CH // _L):
                v = didx_v[j, pl.ds(k * _L, _L)] - base
                ok = (v >= 0) & (v < _NHALF)
                didx_v[j, pl.ds(k * _L, _L)] = jnp.where(
                    ok, v, jnp.full((_L,), _TRASH, jnp.int32))
            return carry

        lax.fori_loop(0, _NCH, tbody, 0)

        pltpu.async_copy(g_hbm.at[sidx_v.at[0]], buf0, sem0)

        def lbody(j2, carry):
            for ph in range(2):
                j = j2 * 2 + ph
                cur, csem = bufs[ph], sems[ph]

                @pl.when(j + 1 < _NCH)
                def _():
                    pltpu.async_copy(g_hbm.at[sidx_v.at[j + 1]],
                                     bufs[1 - ph], sems[1 - ph])

                pltpu.make_async_copy(g_hbm.at[sidx_v.at[j]], cur,
                                      csem).wait()
                pltpu.sync_copy(cur, acc_sp.at[didx_v.at[j]], add=True)
            return carry

        lax.fori_loop(0, _NCH // 2, lbody, 0)
        jt = _NCH - 1
        pltpu.make_async_copy(g_hbm.at[sidx_v.at[jt]], buf0, sem0).wait()
        pltpu.sync_copy(buf0, acc_sp.at[didx_v.at[jt]], add=True)

    plsc.subcore_barrier()
    pltpu.sync_copy(acc_sp.at[pl.ds(start, 312)],
                    acc_out.at[cid, pl.ds(start, 312)])

    @pl.when(s == _NS - 1)
    def _():
        pltpu.sync_copy(acc_sp.at[pl.ds(4992, 16)],
                        acc_out.at[cid, pl.ds(4992, 16)])


def _scatter_accs(g, src3, dst3):
    zeros = jnp.zeros((312, H), jnp.float32)
    return pl.kernel(
        _scat_body,
        out_type=jax.ShapeDtypeStruct((_NC, _ACCR, H), jnp.float32),
        mesh=_MESH,
        compiler_params=_SC_PARAMS,
        scratch_types=[
            pltpu.VMEM((_NCH, _CH), jnp.int32),
            pltpu.VMEM((_NCH, _CH), jnp.int32),
            pltpu.VMEM((_CH, H), jnp.float32),
            pltpu.VMEM((_CH, H), jnp.float32),
            pltpu.VMEM_SHARED((_ACCR, H), jnp.float32),
            pltpu.SemaphoreType.DMA,
            pltpu.SemaphoreType.DMA,
        ],
    )(g, src3, dst3, zeros)


_SCH = 80
_SNCH = _EW // _SCH


def _score_body(ha_hbm, hb_hbm, lsrc_hbm, ldst_hbm, out_hbm,
                sidx_v, didx_v, a0, a1, b0, b1, p0, p1,
                sa0, sa1, sb0, sb1, sp0, sp1):
    w = _wid()
    pltpu.sync_copy(lsrc_hbm.at[w], sidx_v)
    pltpu.sync_copy(ldst_hbm.at[w], didx_v)

    abufs, asems = (a0, a1), (sa0, sa1)
    bbufs, bsems = (b0, b1), (sb0, sb1)
    pbufs, psems = (p0, p1), (sp0, sp1)
    pltpu.async_copy(ha_hbm.at[sidx_v.at[0]], a0, sa0)
    pltpu.async_copy(hb_hbm.at[didx_v.at[0]], b0, sb0)

    def _chunk(j, ca, cb, cp):
        def pbody(p2, carry):
            for q in range(4):
                p = p2 * 4 + q
                acc = ca[p, pl.ds(0, _L)] * cb[p, pl.ds(0, _L)]
                for k in range(1, H // _L):
                    acc = acc + (ca[p, pl.ds(k * _L, _L)]
                                 * cb[p, pl.ds(k * _L, _L)])
                cp[p, pl.ds(0, _L)] = acc
            return carry

        lax.fori_loop(0, _SCH // 4, pbody, 0)

    def lbody(j2, carry):
        for ph in range(2):
            j = j2 * 2 + ph
            ca, cb = abufs[ph], bbufs[ph]
            csa, csb = asems[ph], bsems[ph]
            cp, csp = pbufs[ph], psems[ph]

            @pl.when(j + 1 < _SNCH)
            def _():
                pltpu.async_copy(ha_hbm.at[sidx_v.at[j + 1]],
                                 abufs[1 - ph], asems[1 - ph])
                pltpu.async_copy(hb_hbm.at[didx_v.at[j + 1]],
                                 bbufs[1 - ph], bsems[1 - ph])

            pltpu.make_async_copy(ha_hbm.at[sidx_v.at[j]], ca, csa).wait()
            pltpu.make_async_copy(hb_hbm.at[didx_v.at[j]], cb, csb).wait()

            @pl.when(j >= 2)
            def _():
                pltpu.make_async_copy(
                    cp, out_hbm.at[pl.ds(w * _EW + (j - 2) * _SCH, _SCH)],
                    csp).wait()

            _chunk(j, ca, cb, cp)
            pltpu.async_copy(
                cp, out_hbm.at[pl.ds(w * _EW + j * _SCH, _SCH)], csp)
        return carry

    lax.fori_loop(0, _SNCH // 2, lbody, 0)
    jt = _SNCH - 1
    pltpu.make_async_copy(ha_hbm.at[sidx_v.at[jt]], a0, sa0).wait()
    pltpu.make_async_copy(hb_hbm.at[didx_v.at[jt]], b0, sb0).wait()
    pltpu.make_async_copy(
        p0, out_hbm.at[pl.ds(w * _EW + (jt - 2) * _SCH, _SCH)], sp0).wait()
    _chunk(jt, a0, b0, p0)
    pltpu.async_copy(p0, out_hbm.at[pl.ds(w * _EW + jt * _SCH, _SCH)], sp0)
    pltpu.make_async_copy(
        p0, out_hbm.at[pl.ds(w * _EW + jt * _SCH, _SCH)], sp0).wait()
    pltpu.make_async_copy(
        p1, out_hbm.at[pl.ds(w * _EW + (jt - 1) * _SCH, _SCH)], sp1).wait()


def _score(hA, hB, lsrc3, ldst3):
    return pl.kernel(
        _score_body,
        out_type=jax.ShapeDtypeStruct((EL, _L), jnp.float32),
        mesh=_MESH,
        compiler_params=_SC_PARAMS,
        scratch_types=[
            pltpu.VMEM((_SNCH, _SCH), jnp.int32),
            pltpu.VMEM((_SNCH, _SCH), jnp.int32),
            pltpu.VMEM((_SCH, H), jnp.float32),
            pltpu.VMEM((_SCH, H), jnp.float32),
            pltpu.VMEM((_SCH, H), jnp.float32),
            pltpu.VMEM((_SCH, H), jnp.float32),
            pltpu.VMEM((_SCH, _L), jnp.float32),
            pltpu.VMEM((_SCH, _L), jnp.float32),
            pltpu.SemaphoreType.DMA,
            pltpu.SemaphoreType.DMA,
            pltpu.SemaphoreType.DMA,
            pltpu.SemaphoreType.DMA,
            pltpu.SemaphoreType.DMA,
            pltpu.SemaphoreType.DMA,
        ],
    )(hA, hB, lsrc3, ldst3)


def _red_body(p_ref, c_ref, out_ref):
    out_ref[...] = jnp.sum(p_ref[...], axis=-1, keepdims=True) + c_ref[...]


def _reduce(partials, c):
    blk = 8000
    return pl.pallas_call(
        _red_body,
        out_shape=jax.ShapeDtypeStruct((EL, 1), jnp.float32),
        grid=(EL // blk,),
        in_specs=[
            pl.BlockSpec((blk, _L), lambda i: (i, 0)),
            pl.BlockSpec((1, 1), lambda i: (0, 0)),
        ],
        out_specs=pl.BlockSpec((blk, 1), lambda i: (i, 0)),
    )(partials, c.reshape(1, 1))


def kernel(x, edge_index, edge_label_index, W_pre, b_pre, initial_weight,
           W_ih, W_hh, b_ih, b_hh, gcn_bias, W_post, b_post):
    src3 = edge_index[0].astype(jnp.int32).reshape(_NS, 2, _NCH, _CH)
    dst3 = edge_index[1].astype(jnp.int32).reshape(_NS, 2, _NCH, _CH)
    dst2 = edge_index[1].astype(jnp.int32).reshape(_NW, _EW)
    lsrc3 = edge_label_index[0].astype(jnp.int32).reshape(_NW, _SNCH, _SCH)
    ldst3 = edge_label_index[1].astype(jnp.int32).reshape(_NW, _SNCH, _SCH)
    w = W_post[0] + W_post[1]
    c = b_post[0] + b_post[1]

    hW = _pre(x.astype(jnp.float32), W_pre, b_pre, initial_weight,
              W_ih, W_hh, b_ih, b_hh)
    degp = _deg_partials(dst2)
    g, dinv, selfs = _scale(hW, degp)
    accs = _scatter_accs(g, src3, dst3)
    hA, hB = _finalize(accs, hW, dinv, selfs, gcn_bias, w)
    partials = _score(hA, hB, lsrc3, ldst3)
    return _reduce(partials, c).reshape(EL)


# speedup vs baseline: 2.6366x; 1.0004x over previous
import functools

import jax
import jax.numpy as jnp
from jax import lax
from jax.experimental import pallas as pl
from jax.experimental.pallas import tpu as pltpu
from jax.experimental.pallas import tpu_sc as plsc

N, E, EL, D, H = 10000, 320000, 320000, 128, 128

_NC, _NS, _L = 2, 16, 16
_NW = _NC * _NS
_EW = E // _NW
_CH = 80
_NCH = _EW // _CH
_ECT = E // _NS
_NCHS = _ECT // _CH
_NHALF = N // 2
_ACCR = _NHALF + 8
_TRASH = _NHALF

_MESH = plsc.VectorSubcoreMesh(core_axis_name="c", subcore_axis_name="s",
                               num_cores=_NC, num_subcores=_NS)
_SC_PARAMS = pltpu.CompilerParams(needs_layout_passes=False)


def _wid():
    return lax.axis_index("c") * _NS + lax.axis_index("s")


def _pre_body(x_ref, wpre_ref, bpre_ref, iw_ref, wih_ref, whh_ref,
              bih_ref, bhh_ref, hw_ref):
    iw = iw_ref[...]
    gi = lax.dot_general(iw, wih_ref[...], (((1,), (1,)), ((), ())),
                         preferred_element_type=jnp.float32) + bih_ref[...]
    gh = lax.dot_general(iw, whh_ref[...], (((1,), (1,)), ((), ())),
                         preferred_element_type=jnp.float32) + bhh_ref[...]
    i_r, i_z, i_n = gi[:, :H], gi[:, H:2 * H], gi[:, 2 * H:]
    h_r, h_z, h_n = gh[:, :H], gh[:, H:2 * H], gh[:, 2 * H:]
    r = jax.nn.sigmoid(i_r + h_r)
    z = jax.nn.sigmoid(i_z + h_z)
    n = jnp.tanh(i_n + r * h_n)
    W = (1.0 - z) * n + z * iw
    M = lax.dot_general(wpre_ref[...], W, (((0,), (0,)), ((), ())),
                        preferred_element_type=jnp.float32)
    v = jnp.dot(bpre_ref[...], W, preferred_element_type=jnp.float32)
    hw_ref[...] = jnp.dot(x_ref[...], M,
                          preferred_element_type=jnp.float32) + v


def _pre(x, W_pre, b_pre, iw, W_ih, W_hh, b_ih, b_hh):
    return pl.pallas_call(
        _pre_body,
        out_shape=jax.ShapeDtypeStruct((N, H), jnp.float32),
    )(x, W_pre, b_pre.reshape(1, H), iw, W_ih, W_hh,
      b_ih.reshape(1, 3 * H), b_hh.reshape(1, 3 * H))


def _scale_body(hw_ref, degp_ref, g_ref, dinv_ref, selfs_ref):
    deg = 1.0 + jnp.sum(degp_ref[...], axis=0)
    dinv = lax.rsqrt(deg)
    dinv_ref[...] = dinv[:, None]
    selfs_ref[...] = (1.0 / deg)[:, None]
    g_ref[...] = hw_ref[...] * dinv[:, None]


def _scale(hW, deg_partials):
    return pl.pallas_call(
        _scale_body,
        out_shape=(
            jax.ShapeDtypeStruct((N, H), jnp.float32),
            jax.ShapeDtypeStruct((N, 1), jnp.float32),
            jax.ShapeDtypeStruct((N, 1), jnp.float32),
        ),
    )(hW, deg_partials)


def _fin_body(acc_ref, hw_ref, dinv_ref, selfs_ref, bias_ref, w_ref,
              ha_ref, hb_ref):
    acc = jnp.concatenate([acc_ref[0, :_NHALF], acc_ref[1, :_NHALF]], axis=0)
    h = acc * dinv_ref[...] + hw_ref[...] * selfs_ref[...] + bias_ref[...]
    h = jnp.maximum(h, 0.0)
    hb_ref[...] = h
    ha_ref[...] = h * w_ref[...]


def _finalize(accs, hW, dinv, selfs, gcn_bias, w):
    return pl.pallas_call(
        _fin_body,
        out_shape=(
            jax.ShapeDtypeStruct((N, H), jnp.float32),
            jax.ShapeDtypeStruct((N, H), jnp.float32),
        ),
    )(accs, hW, dinv, selfs, gcn_bias.reshape(1, H), w.reshape(1, H))


def _deg_body(dst_hbm, out_hbm, idx_v, hist_v):
    w = _wid()
    pltpu.sync_copy(dst_hbm.at[w], idx_v)
    zeros = jnp.zeros((_L,), jnp.float32)

    def zbody(i, carry):
        hist_v[pl.ds(i * _L, _L)] = zeros
        return carry

    lax.fori_loop(0, N // _L, zbody, 0)
    ones = jnp.ones((_L,), jnp.float32)

    def cbody(i, carry):
        idx = idx_v[pl.ds(i * _L, _L)]
        plsc.addupdate_scatter(hist_v, [idx], ones)
        return carry

    lax.fori_loop(0, _EW // _L, cbody, 0)
    pltpu.sync_copy(hist_v, out_hbm.at[w])


def _deg_partials(dst2):
    return pl.kernel(
        _deg_body,
        out_type=jax.ShapeDtypeStruct((_NW, N), jnp.float32),
        mesh=_MESH,
        compiler_params=_SC_PARAMS,
        scratch_types=[
            pltpu.VMEM((_EW,), jnp.int32),
            pltpu.VMEM((N,), jnp.float32),
        ],
    )(dst2)


def _scat_body(g_hbm, src_hbm, dst_hbm, zeros_hbm, acc_out,
               sidx_v, didx_v, buf0, buf1, acc_sp, sem0, sem1, ssem0, ssem1):
    cid = lax.axis_index("c")
    s = lax.axis_index("s")

    start = s * 312
    pltpu.sync_copy(zeros_hbm.at[pl.ds(0, 312)], acc_sp.at[pl.ds(start, 312)])

    @pl.when(s == _NS - 1)
    def _():
        pltpu.sync_copy(zeros_hbm.at[pl.ds(0, 16)],
                        acc_sp.at[pl.ds(4992, 16)])

    plsc.subcore_barrier()

    base = cid * _NHALF
    bufs = (buf0, buf1)
    sems = (sem0, sem1)
    ssems = (ssem0, ssem1)

    for hlf in range(2):
        pltpu.sync_copy(src_hbm.at[s, hlf], sidx_v)
        pltpu.sync_copy(dst_hbm.at[s, hlf], didx_v)

        def tbody(j, carry):
            for k in range(_CH // _L):
                v = didx_v[j, pl.ds(k * _L, _L)] - base
                ok = (v >= 0) & (v < _NHALF)
                didx_v[j, pl.ds(k * _L, _L)] = jnp.where(
                    ok, v, jnp.full((_L,), _TRASH, jnp.int32))
            return carry

        lax.fori_loop(0, _NCH, tbody, 0)

        pltpu.async_copy(g_hbm.at[sidx_v.at[0]], buf0, sem0)

        def lbody(j2, carry):
            for ph in range(2):
                j = j2 * 2 + ph
                cur, csem, cssem = bufs[ph], sems[ph], ssems[ph]
                oth, osem, ossem = bufs[1 - ph], sems[1 - ph], ssems[1 - ph]

                @pl.when(j >= 1)
                def _():
                    pltpu.make_async_copy(
                        oth, acc_sp.at[didx_v.at[j - 1]], ossem).wait()

                @pl.when(j + 1 < _NCH)
                def _():
                    pltpu.async_copy(g_hbm.at[sidx_v.at[j + 1]], oth, osem)

                pltpu.make_async_copy(g_hbm.at[sidx_v.at[j]], cur,
                                      csem).wait()
                pltpu.async_copy(cur, acc_sp.at[didx_v.at[j]], cssem,
                                 add=True)
            return carry

        lax.fori_loop(0, _NCH // 2, lbody, 0)
        jt = _NCH - 1
        pltpu.make_async_copy(g_hbm.at[sidx_v.at[jt]], buf0, sem0).wait()
        pltpu.async_copy(buf0, acc_sp.at[didx_v.at[jt]], ssem0, add=True)
        pltpu.make_async_copy(buf1, acc_sp.at[didx_v.at[jt - 1]],
                              ssem1).wait()
        pltpu.make_async_copy(buf0, acc_sp.at[didx_v.at[jt]], ssem0).wait()

    plsc.subcore_barrier()
    pltpu.sync_copy(acc_sp.at[pl.ds(start, 312)],
                    acc_out.at[cid, pl.ds(start, 312)])

    @pl.when(s == _NS - 1)
    def _():
        pltpu.sync_copy(acc_sp.at[pl.ds(4992, 16)],
                        acc_out.at[cid, pl.ds(4992, 16)])


def _scatter_accs(g, src3, dst3):
    zeros = jnp.zeros((312, H), jnp.float32)
    return pl.kernel(
        _scat_body,
        out_type=jax.ShapeDtypeStruct((_NC, _ACCR, H), jnp.float32),
        mesh=_MESH,
        compiler_params=_SC_PARAMS,
        scratch_types=[
            pltpu.VMEM((_NCH, _CH), jnp.int32),
            pltpu.VMEM((_NCH, _CH), jnp.int32),
            pltpu.VMEM((_CH, H), jnp.float32),
            pltpu.VMEM((_CH, H), jnp.float32),
            pltpu.VMEM_SHARED((_ACCR, H), jnp.float32),
            pltpu.SemaphoreType.DMA,
            pltpu.SemaphoreType.DMA,
            pltpu.SemaphoreType.DMA,
            pltpu.SemaphoreType.DMA,
        ],
    )(g, src3, dst3, zeros)


_SCH = 80
_SNCH = _EW // _SCH


def _score_body(ha_hbm, hb_hbm, lsrc_hbm, ldst_hbm, out_hbm,
                sidx_v, didx_v, a0, a1, b0, b1, p0, p1,
                sa0, sa1, sb0, sb1, sp0, sp1):
    w = _wid()
    pltpu.sync_copy(lsrc_hbm.at[w], sidx_v)
    pltpu.sync_copy(ldst_hbm.at[w], didx_v)

    abufs, asems = (a0, a1), (sa0, sa1)
    bbufs, bsems = (b0, b1), (sb0, sb1)
    pbufs, psems = (p0, p1), (sp0, sp1)
    pltpu.async_copy(ha_hbm.at[sidx_v.at[0]], a0, sa0)
    pltpu.async_copy(hb_hbm.at[didx_v.at[0]], b0, sb0)

    def _chunk(j, ca, cb, cp):
        def pbody(p2, carry):
            for q in range(4):
                p = p2 * 4 + q
                acc = ca[p, pl.ds(0, _L)] * cb[p, pl.ds(0, _L)]
                for k in range(1, H // _L):
                    acc = acc + (ca[p, pl.ds(k * _L, _L)]
                                 * cb[p, pl.ds(k * _L, _L)])
                cp[p, pl.ds(0, _L)] = acc
            return carry

        lax.fori_loop(0, _SCH // 4, pbody, 0)

    def lbody(j2, carry):
        for ph in range(2):
            j = j2 * 2 + ph
            ca, cb = abufs[ph], bbufs[ph]
            csa, csb = asems[ph], bsems[ph]
            cp, csp = pbufs[ph], psems[ph]

            @pl.when(j + 1 < _SNCH)
            def _():
                pltpu.async_copy(ha_hbm.at[sidx_v.at[j + 1]],
                                 abufs[1 - ph], asems[1 - ph])
                pltpu.async_copy(hb_hbm.at[didx_v.at[j + 1]],
                                 bbufs[1 - ph], bsems[1 - ph])

            pltpu.make_async_copy(ha_hbm.at[sidx_v.at[j]], ca, csa).wait()
            pltpu.make_async_copy(hb_hbm.at[didx_v.at[j]], cb, csb).wait()

            @pl.when(j >= 2)
            def _():
                pltpu.make_async_copy(
                    cp, out_hbm.at[pl.ds(w * _EW + (j - 2) * _SCH, _SCH)],
                    csp).wait()

            _chunk(j, ca, cb, cp)
            pltpu.async_copy(
                cp, out_hbm.at[pl.ds(w * _EW + j * _SCH, _SCH)], csp)
        return carry

    lax.fori_loop(0, _SNCH // 2, lbody, 0)
    jt = _SNCH - 1
    pltpu.make_async_copy(ha_hbm.at[sidx_v.at[jt]], a0, sa0).wait()
    pltpu.make_async_copy(hb_hbm.at[didx_v.at[jt]], b0, sb0).wait()
    pltpu.make_async_copy(
        p0, out_hbm.at[pl.ds(w * _EW + (jt - 2) * _SCH, _SCH)], sp0).wait()
    _chunk(jt, a0, b0, p0)
    pltpu.async_copy(p0, out_hbm.at[pl.ds(w * _EW + jt * _SCH, _SCH)], sp0)
    pltpu.make_async_copy(
        p0, out_hbm.at[pl.ds(w * _EW + jt * _SCH, _SCH)], sp0).wait()
    pltpu.make_async_copy(
        p1, out_hbm.at[pl.ds(w * _EW + (jt - 1) * _SCH, _SCH)], sp1).wait()


def _score(hA, hB, lsrc3, ldst3):
    return pl.kernel(
        _score_body,
        out_type=jax.ShapeDtypeStruct((EL, _L), jnp.float32),
        mesh=_MESH,
        compiler_params=_SC_PARAMS,
        scratch_types=[
            pltpu.VMEM((_SNCH, _SCH), jnp.int32),
            pltpu.VMEM((_SNCH, _SCH), jnp.int32),
            pltpu.VMEM((_SCH, H), jnp.float32),
            pltpu.VMEM((_SCH, H), jnp.float32),
            pltpu.VMEM((_SCH, H), jnp.float32),
            pltpu.VMEM((_SCH, H), jnp.float32),
            pltpu.VMEM((_SCH, _L), jnp.float32),
            pltpu.VMEM((_SCH, _L), jnp.float32),
            pltpu.SemaphoreType.DMA,
            pltpu.SemaphoreType.DMA,
            pltpu.SemaphoreType.DMA,
            pltpu.SemaphoreType.DMA,
            pltpu.SemaphoreType.DMA,
            pltpu.SemaphoreType.DMA,
        ],
    )(hA, hB, lsrc3, ldst3)


def _red_body(p_ref, c_ref, out_ref):
    out_ref[...] = jnp.sum(p_ref[...], axis=-1, keepdims=True) + c_ref[...]


def _reduce(partials, c):
    blk = 8000
    return pl.pallas_call(
        _red_body,
        out_shape=jax.ShapeDtypeStruct((EL, 1), jnp.float32),
        grid=(EL // blk,),
        in_specs=[
            pl.BlockSpec((blk, _L), lambda i: (i, 0)),
            pl.BlockSpec((1, 1), lambda i: (0, 0)),
        ],
        out_specs=pl.BlockSpec((blk, 1), lambda i: (i, 0)),
    )(partials, c.reshape(1, 1))


def kernel(x, edge_index, edge_label_index, W_pre, b_pre, initial_weight,
           W_ih, W_hh, b_ih, b_hh, gcn_bias, W_post, b_post):
    src3 = edge_index[0].astype(jnp.int32).reshape(_NS, 2, _NCH, _CH)
    dst3 = edge_index[1].astype(jnp.int32).reshape(_NS, 2, _NCH, _CH)
    dst2 = edge_index[1].astype(jnp.int32).reshape(_NW, _EW)
    lsrc3 = edge_label_index[0].astype(jnp.int32).reshape(_NW, _SNCH, _SCH)
    ldst3 = edge_label_index[1].astype(jnp.int32).reshape(_NW, _SNCH, _SCH)
    w = W_post[0] + W_post[1]
    c = b_post[0] + b_post[1]

    hW = _pre(x.astype(jnp.float32), W_pre, b_pre, initial_weight,
              W_ih, W_hh, b_ih, b_hh)
    degp = _deg_partials(dst2)
    g, dinv, selfs = _scale(hW, degp)
    accs = _scatter_accs(g, src3, dst3)
    hA, hB = _finalize(accs, hW, dinv, selfs, gcn_bias, w)
    partials = _score(hA, hB, lsrc3, ldst3)
    return _reduce(partials, c).reshape(EL)
